# Initial kernel scaffold; baseline (speedup 1.0000x reference)
import jax
import jax.numpy as jnp
from jax.experimental import pallas as pl


def kernel(loc_data, conf_data, priors, targets):
    raise NotImplementedError("write your pallas kernel here")


# speedup vs baseline: 36.2637x; 36.2637x over previous
import functools

import jax
import jax.numpy as jnp
from jax import lax
from jax.experimental import pallas as pl
from jax.experimental.pallas import tpu as pltpu

jax.config.update("jax_enable_x64", True)

_THRESHOLD = 0.5
_V0 = 0.1
_V1 = 0.2


def _row_kernel(tgt_ref, conf_ref, loc_ref, pri_ref, mine_ref, stats_ref, *,
                num_obj, num_classes, num_priors, lanes):
    S, L = 8, lanes
    f32 = jnp.float32

    cx = pri_ref[0]
    cy = pri_ref[1]
    w = pri_ref[2]
    h = pri_ref[3]
    px1 = cx - w / 2.0
    py1 = cy - h / 2.0
    px2 = cx + w / 2.0
    py2 = cy + h / 2.0
    area_p = (px2 - px1) * (py2 - py1)

    sub_iota = lax.broadcasted_iota(jnp.int32, (S, L), 0)
    lane_iota = lax.broadcasted_iota(jnp.int32, (S, L), 1)
    p_iota = sub_iota * L + lane_iota
    valid = p_iota < num_priors

    bto = jnp.full((S, L), -1.0, f32)
    bti = jnp.zeros((S, L), jnp.int32)
    tcoords = []
    bp_list = []
    for j in range(num_obj):
        tx1 = tgt_ref[0, 0, 5 * j + 0]
        ty1 = tgt_ref[0, 0, 5 * j + 1]
        tx2 = tgt_ref[0, 0, 5 * j + 2]
        ty2 = tgt_ref[0, 0, 5 * j + 3]
        tlab = tgt_ref[0, 0, 5 * j + 4]
        tcoords.append((tx1, ty1, tx2, ty2, tlab))
        area_t = (tx2 - tx1) * (ty2 - ty1)
        iw = jnp.maximum(jnp.minimum(px2, tx2) - jnp.maximum(px1, tx1), 0.0)
        ih = jnp.maximum(jnp.minimum(py2, ty2) - jnp.maximum(py1, ty1), 0.0)
        inter = iw * ih
        ovl = inter / (area_t + area_p - inter)
        upd = ovl > bto
        bti = jnp.where(upd, j, bti)
        bto = jnp.where(upd, ovl, bto)
        mx = jnp.max(ovl, axis=(0, 1), keepdims=True)
        cand = jnp.where(ovl == mx, p_iota, jnp.int32(2**30))
        bp_list.append(jnp.min(cand, axis=(0, 1), keepdims=True))

    for j in range(num_obj):
        hit = p_iota == bp_list[j]
        bti = jnp.where(hit, j, bti)
        bto = jnp.where(hit, 2.0, bto)

    mx1 = jnp.zeros((S, L), f32)
    my1 = jnp.zeros((S, L), f32)
    mx2 = jnp.zeros((S, L), f32)
    my2 = jnp.zeros((S, L), f32)
    lab = jnp.zeros((S, L), f32)
    for j in range(num_obj):
        sel = bti == j
        tx1, ty1, tx2, ty2, tlab = tcoords[j]
        mx1 = jnp.where(sel, tx1, mx1)
        my1 = jnp.where(sel, ty1, my1)
        mx2 = jnp.where(sel, tx2, mx2)
        my2 = jnp.where(sel, ty2, my2)
        lab = jnp.where(sel, tlab, lab)

    pos = bto >= _THRESHOLD
    posv = pos & valid
    npos = jnp.sum(posv.astype(f32), axis=(0, 1), keepdims=True)

    g_cx = ((mx1 + mx2) / 2.0 - cx) / (_V0 * w)
    g_cy = ((my1 + my2) / 2.0 - cy) / (_V0 * h)
    g_w = jnp.log((mx2 - mx1) / w) / _V1
    g_h = jnp.log((my2 - my1) / h) / _V1
    ll = jnp.zeros((1, 1), f32)
    for c, g in enumerate((g_cx, g_cy, g_w, g_h)):
        d = loc_ref[0, c] - g
        ad = jnp.abs(d)
        sl = jnp.where(ad < 1.0, 0.5 * d * d, ad - 0.5)
        ll = ll + jnp.sum(jnp.where(posv, sl, 0.0), axis=(0, 1), keepdims=True)

    ct = jnp.where(pos, lab + 1.0, 0.0)
    cmax = conf_ref[0, 0]
    for c in range(1, num_classes):
        cmax = jnp.maximum(cmax, conf_ref[0, c])
    sumexp = jnp.zeros((S, L), f32)
    gathered = jnp.zeros((S, L), f32)
    for c in range(num_classes):
        xc = conf_ref[0, c]
        sumexp = sumexp + jnp.exp(xc - cmax)
        gathered = jnp.where(ct == c, xc, gathered)
    ce = jnp.log(sumexp) + cmax - gathered
    pce = jnp.sum(jnp.where(posv, ce, 0.0), axis=(0, 1), keepdims=True)

    mine_ref[0] = jnp.where(valid & (~pos), ce, -1.0)

    s8 = lax.broadcasted_iota(jnp.int32, (8, 128), 0)
    l8 = lax.broadcasted_iota(jnp.int32, (8, 128), 1)
    row0 = s8 == 0
    stats = (jnp.where(row0 & (l8 == 0), npos, 0.0)
             + jnp.where(row0 & (l8 == 1), pce, 0.0)
             + jnp.where(row0 & (l8 == 2), ll, 0.0))
    stats_ref[0] = stats


def _select_kernel(mine_ref, stats_ref, out_ref, *, num_priors, lanes):
    f32 = jnp.float32
    S, L = 8, lanes
    mine = mine_ref[...]
    bits = lax.bitcast_convert_type(mine, jnp.int32)
    B = mine.shape[0]
    st = stats_ref[...]
    npos = st[:, 0:1, 0:1]
    pce = st[:, 0:1, 1:2]
    llr = st[:, 0:1, 2:3]
    k = jnp.minimum(3.0 * npos, jnp.float32(num_priors - 1))

    def body(_, lohi):
        lo, hi = lohi
        mid = lo + ((hi - lo) >> 1)
        cnt = jnp.sum((bits > mid).astype(f32), axis=(1, 2), keepdims=True)
        ok = cnt <= k
        return jnp.where(ok, lo, mid + 1), jnp.where(ok, mid, hi)

    lo0 = jnp.zeros((B, 1, 1), jnp.int32)
    hi0 = jnp.full((B, 1, 1), jnp.int32(0x7F000000), jnp.int32)
    _, thr = lax.fori_loop(0, 31, body, (lo0, hi0))

    gt = bits > thr
    n1 = jnp.sum(gt.astype(f32), axis=(1, 2), keepdims=True)
    eq = bits == thr
    m = k - n1

    p_sub = lax.broadcasted_iota(jnp.int32, (1, S, L), 1)
    p_lane = lax.broadcasted_iota(jnp.int32, (1, S, L), 2)
    p_flat = p_sub * L + p_lane

    def body2(_, lohi):
        lo, hi = lohi
        mid = lo + ((hi - lo) >> 1)
        g = jnp.sum((eq & (p_flat <= mid)).astype(f32), axis=(1, 2),
                    keepdims=True)
        ok = g >= m
        return jnp.where(ok, lo, mid + 1), jnp.where(ok, mid, hi)

    lo0b = jnp.zeros((B, 1, 1), jnp.int32)
    hi0b = jnp.full((B, 1, 1), jnp.int32(S * L - 1), jnp.int32)
    _, tie_idx = lax.fori_loop(0, 15, body2, (lo0b, hi0b))

    selneg = gt | (eq & (p_flat <= tie_idx) & (m > 0))
    negsum = jnp.sum(jnp.where(selneg, mine, 0.0), axis=(1, 2), keepdims=True)

    total_lc = jnp.sum(pce + negsum)
    total_ll = jnp.sum(llr)
    total_np = jnp.sum(npos)

    li = lax.broadcasted_iota(jnp.int32, (1, 128), 1)
    out_ref[...] = (jnp.where(li == 0, total_ll, 0.0)
                    + jnp.where(li == 1, total_lc, 0.0)
                    + jnp.where(li == 2, total_np, 0.0))


def kernel(loc_data, conf_data, priors, targets):
    B, P, C = conf_data.shape
    NO = targets.shape[1]
    Ppad = ((P + 1023) // 1024) * 1024
    S, L = 8, Ppad // 8
    pad = Ppad - P

    conf4 = jnp.pad(conf_data, ((0, 0), (0, pad), (0, 0)))
    conf4 = conf4.transpose(0, 2, 1).reshape(B, C, S, L)
    loc4 = jnp.pad(loc_data, ((0, 0), (0, pad), (0, 0)))
    loc4 = loc4.transpose(0, 2, 1).reshape(B, 4, S, L)
    pad_prior = jnp.tile(
        jnp.array([[-10.0, -10.0, 0.1, 0.1]], dtype=jnp.float32), (pad, 1))
    pri4 = jnp.concatenate([priors, pad_prior], axis=0).T.reshape(4, S, L)
    tgt = targets.reshape(B, 1, NO * 5).astype(jnp.float32)

    row_fn = functools.partial(_row_kernel, num_obj=NO, num_classes=C,
                               num_priors=P, lanes=L)
    mine, stats = pl.pallas_call(
        row_fn,
        grid=(B,),
        in_specs=[
            pl.BlockSpec((1, 1, NO * 5), lambda b: (b, jnp.int32(0), jnp.int32(0)),
                         memory_space=pltpu.SMEM),
            pl.BlockSpec((1, C, S, L), lambda b: (b, jnp.int32(0), jnp.int32(0), jnp.int32(0))),
            pl.BlockSpec((1, 4, S, L), lambda b: (b, jnp.int32(0), jnp.int32(0), jnp.int32(0))),
            pl.BlockSpec((4, S, L), lambda b: (jnp.int32(0), jnp.int32(0), jnp.int32(0))),
        ],
        out_specs=[
            pl.BlockSpec((1, S, L), lambda b: (b, jnp.int32(0), jnp.int32(0))),
            pl.BlockSpec((1, 8, 128), lambda b: (b, jnp.int32(0), jnp.int32(0))),
        ],
        out_shape=[
            jax.ShapeDtypeStruct((B, S, L), jnp.float32),
            jax.ShapeDtypeStruct((B, 8, 128), jnp.float32),
        ],
        compiler_params=pltpu.CompilerParams(
            dimension_semantics=("arbitrary",)),
    )(tgt, conf4, loc4, pri4)

    sel_fn = functools.partial(_select_kernel, num_priors=P, lanes=L)
    out = pl.pallas_call(
        sel_fn,
        in_specs=[
            pl.BlockSpec((B, S, L), lambda: (jnp.int32(0), jnp.int32(0), jnp.int32(0))),
            pl.BlockSpec((B, 8, 128), lambda: (jnp.int32(0), jnp.int32(0), jnp.int32(0))),
        ],
        out_specs=pl.BlockSpec((1, 128), lambda: (jnp.int32(0), jnp.int32(0))),
        out_shape=jax.ShapeDtypeStruct((1, 128), jnp.float32),
    )(mine, stats)

    n64 = out[0, 2].astype(jnp.float64)
    loss_l = out[0, 0].astype(jnp.float64) / n64
    loss_c = out[0, 1].astype(jnp.float64) / n64
    return (loss_l, loss_c)
